# baseline (speedup 1.0000x reference)
import functools

import jax
import jax.numpy as jnp
from jax import lax
from jax.experimental import pallas as pl
from jax.experimental.pallas import tpu as pltpu
from jax.experimental.pallas import tpu_sc as plsc

_SC_CORES = 2
_SC_SUBCORES = 16
_NW = _SC_CORES * _SC_SUBCORES

_GATHER_CHUNK = 128


def _pack_kernel(x_ref, out_ref):
    x = x_ref[...]
    half = x.shape[1] // 2
    ul = lax.bitcast_convert_type(x[:, :half], jnp.uint32)
    uh = lax.bitcast_convert_type(x[:, half:], jnp.uint32)
    rl = lax.shift_right_logical(ul + jnp.uint32(0x8000), jnp.uint32(16))
    rh = (uh + jnp.uint32(0x8000)) & jnp.uint32(0xFFFF0000)
    out_ref[...] = lax.bitcast_convert_type(rl | rh, jnp.float32)


def _pack_rows(x):
    R, D = x.shape
    blk = 2048
    return pl.pallas_call(
        _pack_kernel,
        grid=(R // blk,),
        in_specs=[pl.BlockSpec((blk, D), lambda i: (i, 0))],
        out_specs=pl.BlockSpec((blk, D // 2), lambda i: (i, 0)),
        out_shape=jax.ShapeDtypeStruct((R, D // 2), jnp.float32),
    )(x)


def _sc_gather_rows(table, idx):
    R, D = table.shape
    N = idx.shape[0]
    per_w = N // _NW
    n_chunks = per_w // _GATHER_CHUNK
    assert per_w % _GATHER_CHUNK == 0 and N % (8 * _NW) == 0

    mesh = plsc.VectorSubcoreMesh(core_axis_name="c", subcore_axis_name="s")

    @functools.partial(
        pl.kernel,
        mesh=mesh,
        out_type=jax.ShapeDtypeStruct((N, D), jnp.float32),
        scratch_types=[
            [pltpu.VMEM((_GATHER_CHUNK,), jnp.int32) for _ in range(n_chunks)],
            [pltpu.VMEM((_GATHER_CHUNK, D), jnp.float32) for _ in range(n_chunks)],
            [pltpu.SemaphoreType.DMA for _ in range(n_chunks)],
            [pltpu.SemaphoreType.DMA for _ in range(n_chunks)],
        ],
    )
    def gather_kernel(table_hbm, idx_hbm, out_hbm, idx_vs, rows_vs, gsems, wsems):
        wid = lax.axis_index("s") * _SC_CORES + lax.axis_index("c")
        base = wid * per_w
        gathers = []
        for j in range(n_chunks):
            off = base + j * _GATHER_CHUNK
            pltpu.sync_copy(idx_hbm.at[pl.ds(off, _GATHER_CHUNK)], idx_vs[j])
            gathers.append(
                pltpu.async_copy(table_hbm.at[idx_vs[j]], rows_vs[j], gsems[j]))
        writes = []
        for j in range(n_chunks):
            off = base + j * _GATHER_CHUNK
            gathers[j].wait()
            writes.append(
                pltpu.async_copy(rows_vs[j], out_hbm.at[pl.ds(off, _GATHER_CHUNK)],
                                 wsems[j]))
        for w in writes:
            w.wait()

    return gather_kernel(table, idx)


_BT = 1024


def _mlp_kernel(h_ref, colf_ref, motif_ref, ct_ref, w1h_ref, w1clo_ref,
                w1chi_ref, wm_ref, b1_ref, w2_ref, b2_ref, out_ref):
    z = lax.dot_general(h_ref[...].astype(jnp.bfloat16), w1h_ref[...],
                        (((1,), (1,)), ((), ())),
                        preferred_element_type=jnp.float32)
    u = lax.bitcast_convert_type(colf_ref[...], jnp.uint32)
    lo = lax.bitcast_convert_type(u << 16, jnp.float32).astype(jnp.bfloat16)
    hi = lax.bitcast_convert_type(u & jnp.uint32(0xFFFF0000),
                                  jnp.float32).astype(jnp.bfloat16)
    z += lax.dot_general(lo, w1clo_ref[...], (((1,), (1,)), ((), ())),
                         preferred_element_type=jnp.float32)
    z += lax.dot_general(hi, w1chi_ref[...], (((1,), (1,)), ((), ())),
                         preferred_element_type=jnp.float32)
    z += lax.dot_general(motif_ref[...], wm_ref[:, 0:1],
                         (((1,), (1,)), ((), ())),
                         preferred_element_type=jnp.float32)
    z += b1_ref[...]
    hm = jnp.maximum(z, 0.0)
    logit = jnp.sum(hm * w2_ref[...], axis=1, keepdims=True)
    logit = logit + b2_ref[0, 0]
    valid = ct_ref[...] >= 0
    out_ref[...] = jnp.where(valid, logit, 0.0)


def kernel(hidden_states, column_features, W1, b1, W2, b2, c_t, motif_indicators):
    B, T, H = hidden_states.shape
    _, NC, FD = column_features.shape
    N = B * T

    c_safe = jnp.where(c_t >= 0, c_t, 0)
    flat_idx = (jnp.arange(B, dtype=jnp.int32)[:, None] * NC + c_safe).reshape(N)

    table_pk = _pack_rows(column_features.reshape(B * NC, FD))

    h2 = hidden_states.reshape(N, H)
    motif = motif_indicators.reshape(N, 1).astype(jnp.bfloat16)
    ct2 = c_t.reshape(N, 1)

    W1bf = W1.astype(jnp.bfloat16)
    b1r = b1.reshape(1, H)
    b2r = b2.reshape(1, 1)

    hf = FD // 2
    colf_pk = _sc_gather_rows(table_pk, flat_idx)
    out = pl.pallas_call(
        _mlp_kernel,
        grid=(N // _BT,),
        in_specs=[
            pl.BlockSpec((_BT, H), lambda i: (i, 0)),
            pl.BlockSpec((_BT, hf), lambda i: (i, 0)),
            pl.BlockSpec((_BT, 1), lambda i: (i, 0)),
            pl.BlockSpec((_BT, 1), lambda i: (i, 0)),
            pl.BlockSpec((H, H), lambda i: (0, 0)),
            pl.BlockSpec((H, hf), lambda i: (0, H // hf)),
            pl.BlockSpec((H, hf), lambda i: (0, H // hf + 1)),
            pl.BlockSpec((H, 128), lambda i: (0, (H + FD) // 128)),
            pl.BlockSpec((1, H), lambda i: (0, 0)),
            pl.BlockSpec((1, H), lambda i: (0, 0)),
            pl.BlockSpec((1, 1), lambda i: (0, 0)),
        ],
        out_specs=pl.BlockSpec((_BT, 1), lambda i: (i, 0)),
        out_shape=jax.ShapeDtypeStruct((N, 1), jnp.float32),
    )(h2, colf_pk, motif, ct2, W1bf, W1bf, W1bf, W1bf, b1r, W2, b2r)

    return out.reshape(B, T)


# speedup vs baseline: 1.0215x; 1.0215x over previous
import functools

import jax
import jax.numpy as jnp
from jax import lax
from jax.experimental import pallas as pl
from jax.experimental.pallas import tpu as pltpu
from jax.experimental.pallas import tpu_sc as plsc

_SC_CORES = 2
_SC_SUBCORES = 16
_NW = _SC_CORES * _SC_SUBCORES

_GATHER_CHUNK = 128


def _pack_kernel(x_ref, out_ref):
    x = x_ref[...]
    half = x.shape[1] // 2
    ul = lax.bitcast_convert_type(x[:, :half], jnp.uint32)
    uh = lax.bitcast_convert_type(x[:, half:], jnp.uint32)
    rl = lax.shift_right_logical(ul + jnp.uint32(0x8000), jnp.uint32(16))
    rh = (uh + jnp.uint32(0x8000)) & jnp.uint32(0xFFFF0000)
    out_ref[...] = lax.bitcast_convert_type(rl | rh, jnp.float32)


def _pack_rows(x):
    R, D = x.shape
    blk = 2048
    return pl.pallas_call(
        _pack_kernel,
        grid=(R // blk,),
        in_specs=[pl.BlockSpec((blk, D), lambda i: (i, 0))],
        out_specs=pl.BlockSpec((blk, D // 2), lambda i: (i, 0)),
        out_shape=jax.ShapeDtypeStruct((R, D // 2), jnp.float32),
    )(x)


def _sc_gather_rows(table, idx):
    R, D = table.shape
    N = idx.shape[0]
    per_w = N // _NW
    n_chunks = per_w // _GATHER_CHUNK
    assert per_w % _GATHER_CHUNK == 0 and N % (8 * _NW) == 0

    mesh = plsc.VectorSubcoreMesh(core_axis_name="c", subcore_axis_name="s")

    @functools.partial(
        pl.kernel,
        mesh=mesh,
        out_type=jax.ShapeDtypeStruct((N, D), jnp.float32),
        scratch_types=[
            pltpu.VMEM((_GATHER_CHUNK,), jnp.int32),
            pltpu.VMEM((_GATHER_CHUNK, D), jnp.float32),
            pltpu.SemaphoreType.DMA,
        ],
    )
    def gather_kernel(table_hbm, idx_hbm, out_hbm, idx_v, rows_v, sem):
        wid = lax.axis_index("s") * _SC_CORES + lax.axis_index("c")
        base = wid * per_w
        for j in range(n_chunks):
            off = base + j * _GATHER_CHUNK
            pltpu.sync_copy(idx_hbm.at[pl.ds(off, _GATHER_CHUNK)], idx_v)
            pltpu.async_copy(table_hbm.at[idx_v], rows_v, sem).wait()
            pltpu.sync_copy(rows_v, out_hbm.at[pl.ds(off, _GATHER_CHUNK)])

    return gather_kernel(table, idx)


_BT = 1024


def _mlp_kernel(h_ref, colf_ref, motif_ref, ct_ref, w1h_ref, w1clo_ref,
                w1chi_ref, wm_ref, b1_ref, w2_ref, b2_ref, out_ref):
    z = lax.dot_general(h_ref[...].astype(jnp.bfloat16), w1h_ref[...],
                        (((1,), (1,)), ((), ())),
                        preferred_element_type=jnp.float32)
    u = lax.bitcast_convert_type(colf_ref[...], jnp.uint32)
    lo = lax.bitcast_convert_type(u << 16, jnp.float32).astype(jnp.bfloat16)
    hi = lax.bitcast_convert_type(u & jnp.uint32(0xFFFF0000),
                                  jnp.float32).astype(jnp.bfloat16)
    z += lax.dot_general(lo, w1clo_ref[...], (((1,), (1,)), ((), ())),
                         preferred_element_type=jnp.float32)
    z += lax.dot_general(hi, w1chi_ref[...], (((1,), (1,)), ((), ())),
                         preferred_element_type=jnp.float32)
    z += lax.dot_general(motif_ref[...], wm_ref[:, 0:1],
                         (((1,), (1,)), ((), ())),
                         preferred_element_type=jnp.float32)
    z += b1_ref[...]
    hm = jnp.maximum(z, 0.0)
    logit = jnp.sum(hm * w2_ref[...], axis=1, keepdims=True)
    logit = logit + b2_ref[0, 0]
    valid = ct_ref[...] >= 0
    out_ref[...] = jnp.where(valid, logit, 0.0)


def kernel(hidden_states, column_features, W1, b1, W2, b2, c_t, motif_indicators):
    B, T, H = hidden_states.shape
    _, NC, FD = column_features.shape
    N = B * T

    c_safe = jnp.where(c_t >= 0, c_t, 0)
    flat_idx = (jnp.arange(B, dtype=jnp.int32)[:, None] * NC + c_safe).reshape(N)

    table_pk = _pack_rows(column_features.reshape(B * NC, FD))

    h2 = hidden_states.reshape(N, H)
    motif = motif_indicators.reshape(N, 1).astype(jnp.bfloat16)
    ct2 = c_t.reshape(N, 1)

    W1bf = W1.astype(jnp.bfloat16)
    b1r = b1.reshape(1, H)
    b2r = b2.reshape(1, 1)

    hf = FD // 2
    colf_pk = _sc_gather_rows(table_pk, flat_idx)
    out = pl.pallas_call(
        _mlp_kernel,
        grid=(N // _BT,),
        in_specs=[
            pl.BlockSpec((_BT, H), lambda i: (i, 0)),
            pl.BlockSpec((_BT, hf), lambda i: (i, 0)),
            pl.BlockSpec((_BT, 1), lambda i: (i, 0)),
            pl.BlockSpec((_BT, 1), lambda i: (i, 0)),
            pl.BlockSpec((H, H), lambda i: (0, 0)),
            pl.BlockSpec((H, hf), lambda i: (0, H // hf)),
            pl.BlockSpec((H, hf), lambda i: (0, H // hf + 1)),
            pl.BlockSpec((H, 128), lambda i: (0, (H + FD) // 128)),
            pl.BlockSpec((1, H), lambda i: (0, 0)),
            pl.BlockSpec((1, H), lambda i: (0, 0)),
            pl.BlockSpec((1, 1), lambda i: (0, 0)),
        ],
        out_specs=pl.BlockSpec((_BT, 1), lambda i: (i, 0)),
        out_shape=jax.ShapeDtypeStruct((N, 1), jnp.float32),
    )(h2, colf_pk, motif, ct2, W1bf, W1bf, W1bf, W1bf, b1r, W2, b2r)

    return out.reshape(B, T)
